# baseline (speedup 1.0000x reference)
import functools

import jax
import jax.numpy as jnp
from jax import lax
from jax.experimental import pallas as pl
from jax.experimental.pallas import tpu as pltpu
from jax.experimental.pallas import tpu_sc as plsc

N0, N1, N2 = 100000, 25000, 6250
N2P = 6400
NE0, NE1, NE2 = 1600000, 400000, 100000
K = 6
NUM_CLASSES = 40

NCORES, NSUB = 2, 16


def _sc_mesh():
    return plsc.VectorSubcoreMesh(
        core_axis_name="c", subcore_axis_name="s",
        num_cores=NCORES, num_subcores=NSUB)


_SC_PARAMS = pltpu.CompilerParams(use_tc_tiling_on_sc=False)


def _cdiv(a, b):
    return -(-a // b)


def _make_prop_sc(n_pad, F, NE, W, ZC, idx_scale=1, ZZ=200):
    assert NE % W == 0 and W % 8 == 0 and n_pad % ZC == 0 and ZC % 8 == 0
    assert ZC <= W and n_pad % ZZ == 0 and ZZ % 8 == 0
    nwin = NE // W
    nwin_pc = _cdiv(nwin, NCORES)
    trips = _cdiv(nwin_pc, NSUB)
    nzc = n_pad // ZC
    ztrips = _cdiv(nzc, NSUB)
    nzz = n_pad // ZZ
    zztrips = _cdiv(nzz, NSUB)

    @functools.partial(
        pl.kernel,
        out_type=jax.ShapeDtypeStruct((NCORES, n_pad, F), jnp.float32),
        mesh=_sc_mesh(),
        compiler_params=_SC_PARAMS,
        scratch_types=[
            pltpu.VMEM((W,), jnp.int32),
            pltpu.VMEM((W,), jnp.int32),
            pltpu.VMEM((W, F), jnp.float32),
            pltpu.VMEM((ZZ, F), jnp.float32),
            pltpu.VMEM_SHARED((n_pad, F), jnp.float32),
            pltpu.SemaphoreType.DMA,
        ],
    )
    def prop(u_hbm, src_hbm, dst_hbm, z_hbm, part_hbm, idx_s, idx_d, rows,
             zb, ysh, sem):
        c = lax.axis_index("c")
        s = lax.axis_index("s")
        pltpu.sync_copy(z_hbm, zb)
        for t in range(zztrips):
            j = t * NSUB + s

            @pl.when(j < nzz)
            def _():
                pltpu.sync_copy(zb, ysh.at[pl.ds(j * ZZ, ZZ), :])
        plsc.subcore_barrier()
        for t in range(trips):
            wloc = t * NSUB + s
            g = c * nwin_pc + wloc

            @pl.when((wloc < nwin_pc) & (g < nwin))
            def _():
                off = g * W
                pltpu.sync_copy(src_hbm.at[pl.ds(off, W)], idx_s)
                if idx_scale != 1:
                    def obody(i, carry):
                        sl16 = pl.ds(i * 16, 16)
                        idx_s[sl16] = idx_s[sl16] * idx_scale
                        return carry

                    lax.fori_loop(0, W // 16, obody, 0)
                pltpu.async_copy(u_hbm.at[idx_s], rows, sem).wait()
                pltpu.sync_copy(dst_hbm.at[pl.ds(off, W)], idx_d)
                pltpu.sync_copy(rows, ysh.at[idx_d], add=True)
        plsc.subcore_barrier()
        for t in range(ztrips):
            j = t * NSUB + s

            @pl.when(j < nzc)
            def _():
                sl = pl.ds(j * ZC, ZC)
                pltpu.sync_copy(ysh.at[sl, :], rows.at[pl.ds(0, ZC), :])
                pltpu.sync_copy(rows.at[pl.ds(0, ZC), :],
                                part_hbm.at[c, sl, :])

    return prop


def _make_deg_sc(n_pad, NE, W, ZC):
    assert NE % W == 0 and W % 16 == 0 and n_pad % ZC == 0 and ZC % 16 == 0
    nwin = NE // W
    nwin_pc = _cdiv(nwin, NCORES)
    trips = _cdiv(nwin_pc, NSUB)
    nzc = n_pad // ZC
    ztrips = _cdiv(nzc, NSUB)

    @functools.partial(
        pl.kernel,
        out_type=jax.ShapeDtypeStruct((NCORES * n_pad,), jnp.float32),
        mesh=_sc_mesh(),
        compiler_params=_SC_PARAMS,
        scratch_types=[
            pltpu.VMEM((W,), jnp.int32),
            pltpu.VMEM((W,), jnp.float32),
            pltpu.VMEM((ZC,), jnp.float32),
            pltpu.VMEM_SHARED((n_pad,), jnp.float32),
        ],
    )
    def deg(dst_hbm, part_hbm, idx_d, ones_v, zv, ysh):
        c = lax.axis_index("c")
        s = lax.axis_index("s")

        def fill(i, carry):
            ones_v[pl.ds(i * 16, 16)] = jnp.ones((16,), jnp.float32)
            return carry

        lax.fori_loop(0, W // 16, fill, 0)

        def zfill(i, carry):
            zv[pl.ds(i * 16, 16)] = jnp.zeros((16,), jnp.float32)
            return carry

        lax.fori_loop(0, ZC // 16, zfill, 0)
        for t in range(ztrips):
            j = t * NSUB + s

            @pl.when(j < nzc)
            def _():
                pltpu.sync_copy(zv, ysh.at[pl.ds(j * ZC, ZC)])
        plsc.subcore_barrier()
        for t in range(trips):
            wloc = t * NSUB + s
            g = c * nwin_pc + wloc

            @pl.when((wloc < nwin_pc) & (g < nwin))
            def _():
                pltpu.sync_copy(dst_hbm.at[pl.ds(g * W, W)], idx_d)
                pltpu.sync_copy(ones_v, ysh.at[idx_d], add=True)
        plsc.subcore_barrier()
        for t in range(ztrips):
            j = t * NSUB + s

            @pl.when(j < nzc)
            def _():
                pltpu.sync_copy(ysh.at[pl.ds(j * ZC, ZC)], zv)
                pltpu.sync_copy(zv,
                                part_hbm.at[pl.ds(c * n_pad + j * ZC, ZC)])

    return deg


def _make_pool_sc(n_in, F, n_out_pad, W, ZC):
    assert n_in % W == 0 and W % 8 == 0 and n_out_pad % ZC == 0
    assert ZC <= W and ZC % 8 == 0 and F % 16 == 0
    nwin = n_in // W
    nwin_pc = _cdiv(nwin, NCORES)
    trips = _cdiv(nwin_pc, NSUB)
    nzc = n_out_pad // ZC
    ztrips = _cdiv(nzc, NSUB)

    @functools.partial(
        pl.kernel,
        out_type=jax.ShapeDtypeStruct((NCORES, n_out_pad, F), jnp.float32),
        mesh=_sc_mesh(),
        compiler_params=_SC_PARAMS,
        scratch_types=[
            pltpu.VMEM((W,), jnp.int32),
            pltpu.VMEM((W, F), jnp.float32),
            pltpu.VMEM_SHARED((n_out_pad, F), jnp.float32),
        ],
    )
    def pool(x_hbm, rows_hbm, part_hbm, idx_d, rows, ysh):
        c = lax.axis_index("c")
        s = lax.axis_index("s")
        nfc = F // 16

        def zfill(t, carry):
            rows[t // nfc, pl.ds((t % nfc) * 16, 16)] = jnp.zeros(
                (16,), jnp.float32)
            return carry

        lax.fori_loop(0, ZC * nfc, zfill, 0)
        for t in range(ztrips):
            j = t * NSUB + s

            @pl.when(j < nzc)
            def _():
                pltpu.sync_copy(rows.at[pl.ds(0, ZC), :],
                                ysh.at[pl.ds(j * ZC, ZC), :])
        plsc.subcore_barrier()
        for t in range(trips):
            wloc = t * NSUB + s
            g = c * nwin_pc + wloc

            @pl.when((wloc < nwin_pc) & (g < nwin))
            def _():
                off = g * W
                pltpu.sync_copy(x_hbm.at[pl.ds(off, W), :], rows)
                pltpu.sync_copy(rows_hbm.at[pl.ds(off, W)], idx_d)
                pltpu.sync_copy(rows, ysh.at[idx_d], add=True)
        plsc.subcore_barrier()
        for t in range(ztrips):
            j = t * NSUB + s

            @pl.when(j < nzc)
            def _():
                sl = pl.ds(j * ZC, ZC)
                pltpu.sync_copy(ysh.at[sl, :], rows.at[pl.ds(0, ZC), :])
                pltpu.sync_copy(rows.at[pl.ds(0, ZC), :],
                                part_hbm.at[c, sl, :])

    return pool


def _make_mul_sc(n_pad, F, ZC=200):
    assert n_pad % ZC == 0 and ZC % 8 == 0 and F % 16 == 0
    NW = NCORES * NSUB
    nzc = n_pad // ZC
    trips = _cdiv(nzc, NW)
    nfc = F // 16

    @functools.partial(
        pl.kernel,
        out_type=jax.ShapeDtypeStruct((n_pad, F), jnp.float32),
        mesh=_sc_mesh(),
        compiler_params=_SC_PARAMS,
        scratch_types=[
            pltpu.VMEM((ZC, F), jnp.float32),
            pltpu.VMEM((ZC, F), jnp.float32),
        ],
    )
    def mul(tx_hbm, dinvf_hbm, u_hbm, txv, dvv):
        c = lax.axis_index("c")
        s = lax.axis_index("s")
        wid = s * NCORES + c
        for t in range(trips):
            j = t * NW + wid

            @pl.when(j < nzc)
            def _():
                sl = pl.ds(j * ZC, ZC)
                pltpu.sync_copy(tx_hbm.at[sl, :], txv)
                pltpu.sync_copy(dinvf_hbm.at[sl, :], dvv)

                def body(i, carry):
                    r = i // nfc
                    cc = (i % nfc) * 16
                    txv[r, pl.ds(cc, 16)] = (txv[r, pl.ds(cc, 16)] *
                                             dvv[r, pl.ds(cc, 16)])
                    return carry

                lax.fori_loop(0, ZC * nfc, body, 0)
                pltpu.sync_copy(txv, u_hbm.at[sl, :])

    return mul


def _make_combine_sc(n_rows, F, alpha, beta, ZC=200):
    assert n_rows % ZC == 0 and ZC % 8 == 0 and F % 16 == 0
    NW = NCORES * NSUB
    nzc = n_rows // ZC
    trips = _cdiv(nzc, NW)
    nfc = F // 16

    @functools.partial(
        pl.kernel,
        out_type=[
            jax.ShapeDtypeStruct((n_rows, F), jnp.float32),
            jax.ShapeDtypeStruct((n_rows, F), jnp.float32),
        ],
        mesh=_sc_mesh(),
        compiler_params=_SC_PARAMS,
        scratch_types=[
            pltpu.VMEM((ZC, F), jnp.float32),
            pltpu.VMEM((ZC, F), jnp.float32),
            pltpu.VMEM((ZC, F), jnp.float32),
            pltpu.VMEM((ZC, F), jnp.float32),
        ],
    )
    def comb(p0_hbm, p1_hbm, dinvf_hbm, sub_hbm, tx_hbm, u_hbm,
             pa, pb, dv, sv):
        c = lax.axis_index("c")
        s = lax.axis_index("s")
        wid = s * NCORES + c
        for t in range(trips):
            j = t * NW + wid

            @pl.when(j < nzc)
            def _():
                sl = pl.ds(j * ZC, ZC)
                pltpu.sync_copy(p0_hbm.at[sl, :], pa)
                pltpu.sync_copy(p1_hbm.at[sl, :], pb)
                pltpu.sync_copy(dinvf_hbm.at[sl, :], dv)
                pltpu.sync_copy(sub_hbm.at[sl, :], sv)

                def body(i, carry):
                    r = i // nfc
                    cc = pl.ds((i % nfc) * 16, 16)
                    tx = (alpha * dv[r, cc] * (pa[r, cc] + pb[r, cc]) +
                          beta * sv[r, cc])
                    pa[r, cc] = tx
                    pb[r, cc] = dv[r, cc] * tx
                    return carry

                lax.fori_loop(0, ZC * nfc, body, 0)
                pltpu.sync_copy(pa, tx_hbm.at[sl, :])
                pltpu.sync_copy(pb, u_hbm.at[sl, :])

    return comb


def _pre_tc(deg_part, F, BN):
    n = deg_part.shape[1]
    assert n % BN == 0

    def body(dp_ref, dinvf_ref):
        deg = dp_ref[0] + dp_ref[1]
        dinv = jnp.where(deg > 0.0, lax.rsqrt(jnp.maximum(deg, 1e-20)), 0.0)
        dinvf_ref[...] = jnp.broadcast_to(dinv, (BN, F))

    return pl.pallas_call(
        body,
        grid=(n // BN,),
        in_specs=[pl.BlockSpec((NCORES, BN, 1), lambda i: (0, i, 0))],
        out_specs=pl.BlockSpec((BN, F), lambda i: (i, 0)),
        out_shape=jax.ShapeDtypeStruct((n, F), jnp.float32),
    )(deg_part)


def _sum_tc(part, BN):
    _, n, F = part.shape
    assert n % BN == 0

    def body(p_ref, x_ref):
        x_ref[...] = p_ref[0] + p_ref[1]

    return pl.pallas_call(
        body,
        grid=(n // BN,),
        in_specs=[pl.BlockSpec((NCORES, BN, F), lambda i: (0, i, 0))],
        out_specs=pl.BlockSpec((BN, F), lambda i: (i, 0)),
        out_shape=jax.ShapeDtypeStruct((n, F), jnp.float32),
    )(part)


def _cheb_matmul_tc(txs, Wmat, b, vals, relu, BN):
    n, Fin = txs[0].shape
    Fout = Wmat.shape[2]
    assert n % BN == 0
    nv = 1 if vals is not None else 0

    def body(*refs):
        tx_refs = refs[:K]
        w_ref = refs[K]
        b_ref = refs[K + 1]
        v_ref = refs[K + 2] if nv else None
        o_ref = refs[K + 2 + nv]
        acc = jnp.zeros((BN, Fout), jnp.float32) + b_ref[...]
        for k in range(K):
            acc = acc + jnp.dot(tx_refs[k][...], w_ref[k],
                                preferred_element_type=jnp.float32)
        if relu:
            acc = jnp.maximum(acc, 0.0)
        if nv:
            acc = acc * v_ref[...]
        o_ref[...] = acc

    in_specs = [pl.BlockSpec((BN, Fin), lambda i: (i, 0)) for _ in range(K)]
    in_specs.append(pl.BlockSpec((K, Fin, Fout), lambda i: (0, 0, 0)))
    in_specs.append(pl.BlockSpec((1, Fout), lambda i: (0, 0)))
    args = list(txs) + [Wmat, b]
    if nv:
        in_specs.append(pl.BlockSpec((BN, 1), lambda i: (i, 0)))
        args.append(vals)
    return pl.pallas_call(
        body,
        grid=(n // BN,),
        in_specs=in_specs,
        out_specs=pl.BlockSpec((BN, Fout), lambda i: (i, 0)),
        out_shape=jax.ShapeDtypeStruct((n, Fout), jnp.float32),
    )(*args)


def _final_tc(Wl, xf, bl, BC):
    M = Wl.shape[1]
    assert M % BC == 0

    def body(w_ref, x_ref, b_ref, o_ref):
        i = pl.program_id(0)

        @pl.when(i == 0)
        def _():
            o_ref[...] = b_ref[...]

        o_ref[...] += jnp.dot(w_ref[...], x_ref[...],
                              preferred_element_type=jnp.float32)

    return pl.pallas_call(
        body,
        grid=(M // BC,),
        in_specs=[
            pl.BlockSpec((NUM_CLASSES, BC), lambda i: (0, i)),
            pl.BlockSpec((BC, 1), lambda i: (i, 0)),
            pl.BlockSpec((NUM_CLASSES, 1), lambda i: (0, 0)),
        ],
        out_specs=pl.BlockSpec((NUM_CLASSES, 1), lambda i: (0, 0)),
        out_shape=jax.ShapeDtypeStruct((NUM_CLASSES, 1), jnp.float32),
    )(Wl, xf, bl)


def kernel(pos, edge_index, E1_index, E2_index, D0_rows, D0_cols, D0_vals,
           D1_rows, D1_cols, D1_vals, W0, b0, W1, b1, W2, b2, Wl, bl):
    f32 = jnp.float32
    prop0 = _make_prop_sc(N0, 8, NE0, 2000, 1000)
    prop1 = _make_prop_sc(N1, 32, NE1, 2000, 1000)
    prop2 = _make_prop_sc(N2P, 64, NE2, 1000, 200)
    deg0 = _make_deg_sc(N0, NE0, 2000, 400)
    deg1 = _make_deg_sc(25600, NE1, 2000, 400)
    deg2 = _make_deg_sc(N2P, NE2, 2000, 400)
    pool0 = _make_pool_sc(N0, 32, N1, 2000, 1000)
    pool1 = _make_pool_sc(N1, 64, N2P, 1000, 200)
    mul0 = _make_mul_sc(N0 // 2, 16)
    mul1 = _make_mul_sc(N1, 32)
    mul2 = _make_mul_sc(N2P, 64)
    cmb0a = _make_combine_sc(N0 // 2, 16, -1.0, 0.0)
    cmb0b = _make_combine_sc(N0 // 2, 16, -2.0, -1.0)
    cmb1a = _make_combine_sc(N1, 32, -1.0, 0.0)
    cmb1b = _make_combine_sc(N1, 32, -2.0, -1.0)
    cmb2a = _make_combine_sc(N2P, 64, -1.0, 0.0)
    cmb2b = _make_combine_sc(N2P, 64, -2.0, -1.0)
    z8 = jnp.zeros((200, 8), f32)
    z32 = jnp.zeros((200, 32), f32)
    z64 = jnp.zeros((200, 64), f32)
    ident = lambda a: a
    vw0 = lambda a: a.reshape(N0 // 2, 16)
    uv0 = lambda a: a.reshape(N0, 8)

    def run_layer(x, src, dst, prop_fn, mul_fn, comb1, comb2, vw, uv, z,
                  deg_part, Wmat, bias, vals, relu, BN):
        Fd = x.shape[1]
        dinvF = _pre_tc(deg_part, Fd, BN)
        dFv = vw(dinvF)
        txs = [x]
        u = mul_fn(vw(x), dFv)
        for k in range(1, K):
            part = prop_fn(uv(u), src, dst, z)
            sub = vw(x) if k == 1 else vw(txs[-2])
            comb = comb1 if k == 1 else comb2
            tx_v, u = comb(vw(part[0]), vw(part[1]), dFv, sub)
            txs.append(uv(tx_v))
        b2d = bias.reshape(1, -1)
        return _cheb_matmul_tc(txs, Wmat, b2d, vals, relu, BN)

    x0 = jnp.concatenate([pos, jnp.zeros((N0, 5), f32)], axis=1)
    W0p = jnp.concatenate([W0, jnp.zeros((K, 5, 32), f32)], axis=1)
    src0 = edge_index[0].astype(jnp.int32)
    dst0 = edge_index[1].astype(jnp.int32)
    dpart0 = deg0(dst0).reshape(NCORES, N0, 1)
    xv0 = run_layer(x0, src0, dst0, prop0, mul0, cmb0a, cmb0b, vw0, uv0,
                    z8, dpart0, W0p, b0, D0_vals.reshape(N0, 1), True, 1000)
    ppart0 = pool0(xv0, D0_rows.astype(jnp.int32))
    x1 = _sum_tc(ppart0, 1000)
    src1 = E1_index[0].astype(jnp.int32)
    dst1 = E1_index[1].astype(jnp.int32)
    dpart1 = deg1(dst1).reshape(NCORES, 25600, 1)[:, :N1]
    xv1 = run_layer(x1, src1, dst1, prop1, mul1, cmb1a, cmb1b, ident, ident,
                    z32, dpart1, W1, b1, D1_vals.reshape(N1, 1), True, 1000)
    ppart1 = pool1(xv1, D1_rows.astype(jnp.int32))
    x2 = _sum_tc(ppart1, 1280)
    src2 = E2_index[0].astype(jnp.int32)
    dst2 = E2_index[1].astype(jnp.int32)
    dpart2 = deg2(dst2).reshape(NCORES, N2P, 1)
    x3 = run_layer(x2, src2, dst2, prop2, mul2, cmb2a, cmb2b, ident, ident,
                   z64, dpart2, W2, b2, None, False, 1280)
    xf = x3[:N2].reshape(N2 * 64, 1)
    out = _final_tc(Wl, xf, bl.reshape(NUM_CLASSES, 1), 16000)
    return out.reshape(NUM_CLASSES)


# speedup vs baseline: 1.1317x; 1.1317x over previous
import functools

import jax
import jax.numpy as jnp
from jax import lax
from jax.experimental import pallas as pl
from jax.experimental.pallas import tpu as pltpu
from jax.experimental.pallas import tpu_sc as plsc

N0, N1, N2 = 100000, 25000, 6250
N2P = 6400
NE0, NE1, NE2 = 1600000, 400000, 100000
K = 6
NUM_CLASSES = 40

NCORES, NSUB = 2, 16


def _sc_mesh():
    return plsc.VectorSubcoreMesh(
        core_axis_name="c", subcore_axis_name="s",
        num_cores=NCORES, num_subcores=NSUB)


_SC_PARAMS = pltpu.CompilerParams(use_tc_tiling_on_sc=False)


def _cdiv(a, b):
    return -(-a // b)


def _make_prop_sc(n_pad, F, NE, W, ZC, idx_scale=1, ZZ=200):
    assert NE % W == 0 and W % 8 == 0 and n_pad % ZC == 0 and ZC % 8 == 0
    assert ZC <= W and n_pad % ZZ == 0 and ZZ % 8 == 0
    nwin = NE // W
    nwin_pc = _cdiv(nwin, NCORES)
    trips = _cdiv(nwin_pc, NSUB)
    nzc = n_pad // ZC
    ztrips = _cdiv(nzc, NSUB)
    nzz = n_pad // ZZ
    zztrips = _cdiv(nzz, NSUB)

    @functools.partial(
        pl.kernel,
        out_type=jax.ShapeDtypeStruct((NCORES, n_pad, F), jnp.float32),
        mesh=_sc_mesh(),
        compiler_params=_SC_PARAMS,
        scratch_types=[
            pltpu.VMEM((W,), jnp.int32),
            pltpu.VMEM((W,), jnp.int32),
            pltpu.VMEM((W, F), jnp.float32),
            pltpu.VMEM((ZZ, F), jnp.float32),
            pltpu.VMEM_SHARED((n_pad, F), jnp.float32),
            pltpu.SemaphoreType.DMA,
        ],
    )
    def prop(u_hbm, src_hbm, dst_hbm, z_hbm, part_hbm, idx_s, idx_d, rows,
             zb, ysh, sem):
        c = lax.axis_index("c")
        s = lax.axis_index("s")
        pltpu.sync_copy(z_hbm, zb)
        for t in range(zztrips):
            j = t * NSUB + s

            @pl.when(j < nzz)
            def _():
                pltpu.sync_copy(zb, ysh.at[pl.ds(j * ZZ, ZZ), :])
        plsc.subcore_barrier()
        for t in range(trips):
            wloc = t * NSUB + s
            g = c * nwin_pc + wloc

            @pl.when((wloc < nwin_pc) & (g < nwin))
            def _():
                off = g * W
                pltpu.sync_copy(src_hbm.at[pl.ds(off, W)], idx_s)
                if idx_scale != 1:
                    def obody(i, carry):
                        sl16 = pl.ds(i * 16, 16)
                        idx_s[sl16] = idx_s[sl16] * idx_scale
                        return carry

                    lax.fori_loop(0, W // 16, obody, 0)
                pltpu.async_copy(u_hbm.at[idx_s], rows, sem).wait()
                pltpu.sync_copy(dst_hbm.at[pl.ds(off, W)], idx_d)
                pltpu.sync_copy(rows, ysh.at[idx_d], add=True)
        plsc.subcore_barrier()
        for t in range(ztrips):
            j = t * NSUB + s

            @pl.when(j < nzc)
            def _():
                sl = pl.ds(j * ZC, ZC)
                pltpu.sync_copy(ysh.at[sl, :], rows.at[pl.ds(0, ZC), :])
                pltpu.sync_copy(rows.at[pl.ds(0, ZC), :],
                                part_hbm.at[c, sl, :])

    return prop


def _make_deg_sc(n_pad, NE, W, ZC):
    assert NE % W == 0 and W % 16 == 0 and n_pad % ZC == 0 and ZC % 16 == 0
    nwin = NE // W
    nwin_pc = _cdiv(nwin, NCORES)
    trips = _cdiv(nwin_pc, NSUB)
    nzc = n_pad // ZC
    ztrips = _cdiv(nzc, NSUB)

    @functools.partial(
        pl.kernel,
        out_type=jax.ShapeDtypeStruct((NCORES * n_pad,), jnp.float32),
        mesh=_sc_mesh(),
        compiler_params=_SC_PARAMS,
        scratch_types=[
            pltpu.VMEM((W,), jnp.int32),
            pltpu.VMEM((W,), jnp.float32),
            pltpu.VMEM((ZC,), jnp.float32),
            pltpu.VMEM_SHARED((n_pad,), jnp.float32),
        ],
    )
    def deg(dst_hbm, part_hbm, idx_d, ones_v, zv, ysh):
        c = lax.axis_index("c")
        s = lax.axis_index("s")

        def fill(i, carry):
            ones_v[pl.ds(i * 16, 16)] = jnp.ones((16,), jnp.float32)
            return carry

        lax.fori_loop(0, W // 16, fill, 0)

        def zfill(i, carry):
            zv[pl.ds(i * 16, 16)] = jnp.zeros((16,), jnp.float32)
            return carry

        lax.fori_loop(0, ZC // 16, zfill, 0)
        for t in range(ztrips):
            j = t * NSUB + s

            @pl.when(j < nzc)
            def _():
                pltpu.sync_copy(zv, ysh.at[pl.ds(j * ZC, ZC)])
        plsc.subcore_barrier()
        for t in range(trips):
            wloc = t * NSUB + s
            g = c * nwin_pc + wloc

            @pl.when((wloc < nwin_pc) & (g < nwin))
            def _():
                pltpu.sync_copy(dst_hbm.at[pl.ds(g * W, W)], idx_d)
                pltpu.sync_copy(ones_v, ysh.at[idx_d], add=True)
        plsc.subcore_barrier()
        for t in range(ztrips):
            j = t * NSUB + s

            @pl.when(j < nzc)
            def _():
                pltpu.sync_copy(ysh.at[pl.ds(j * ZC, ZC)], zv)
                pltpu.sync_copy(zv,
                                part_hbm.at[pl.ds(c * n_pad + j * ZC, ZC)])

    return deg


def _make_pool_sc(n_in, F, n_out_pad, W, ZC):
    assert n_in % W == 0 and W % 8 == 0 and n_out_pad % ZC == 0
    assert ZC <= W and ZC % 8 == 0 and F % 16 == 0
    nwin = n_in // W
    nwin_pc = _cdiv(nwin, NCORES)
    trips = _cdiv(nwin_pc, NSUB)
    nzc = n_out_pad // ZC
    ztrips = _cdiv(nzc, NSUB)

    @functools.partial(
        pl.kernel,
        out_type=jax.ShapeDtypeStruct((NCORES, n_out_pad, F), jnp.float32),
        mesh=_sc_mesh(),
        compiler_params=_SC_PARAMS,
        scratch_types=[
            pltpu.VMEM((W,), jnp.int32),
            pltpu.VMEM((W, F), jnp.float32),
            pltpu.VMEM_SHARED((n_out_pad, F), jnp.float32),
        ],
    )
    def pool(x_hbm, rows_hbm, part_hbm, idx_d, rows, ysh):
        c = lax.axis_index("c")
        s = lax.axis_index("s")
        nfc = F // 16

        def zfill(t, carry):
            rows[t // nfc, pl.ds((t % nfc) * 16, 16)] = jnp.zeros(
                (16,), jnp.float32)
            return carry

        lax.fori_loop(0, ZC * nfc, zfill, 0)
        for t in range(ztrips):
            j = t * NSUB + s

            @pl.when(j < nzc)
            def _():
                pltpu.sync_copy(rows.at[pl.ds(0, ZC), :],
                                ysh.at[pl.ds(j * ZC, ZC), :])
        plsc.subcore_barrier()
        for t in range(trips):
            wloc = t * NSUB + s
            g = c * nwin_pc + wloc

            @pl.when((wloc < nwin_pc) & (g < nwin))
            def _():
                off = g * W
                pltpu.sync_copy(x_hbm.at[pl.ds(off, W), :], rows)
                pltpu.sync_copy(rows_hbm.at[pl.ds(off, W)], idx_d)
                pltpu.sync_copy(rows, ysh.at[idx_d], add=True)
        plsc.subcore_barrier()
        for t in range(ztrips):
            j = t * NSUB + s

            @pl.when(j < nzc)
            def _():
                sl = pl.ds(j * ZC, ZC)
                pltpu.sync_copy(ysh.at[sl, :], rows.at[pl.ds(0, ZC), :])
                pltpu.sync_copy(rows.at[pl.ds(0, ZC), :],
                                part_hbm.at[c, sl, :])

    return pool


def _make_mul_sc(n_pad, F, ZC=200):
    assert n_pad % ZC == 0 and ZC % 8 == 0 and F % 16 == 0
    NW = NCORES * NSUB
    nzc = n_pad // ZC
    trips = _cdiv(nzc, NW)
    nfc = F // 16

    @functools.partial(
        pl.kernel,
        out_type=jax.ShapeDtypeStruct((n_pad, F), jnp.float32),
        mesh=_sc_mesh(),
        compiler_params=_SC_PARAMS,
        scratch_types=[
            pltpu.VMEM((ZC, F), jnp.float32),
            pltpu.VMEM((ZC, F), jnp.float32),
        ],
    )
    def mul(tx_hbm, dinvf_hbm, u_hbm, txv, dvv):
        c = lax.axis_index("c")
        s = lax.axis_index("s")
        wid = s * NCORES + c
        for t in range(trips):
            j = t * NW + wid

            @pl.when(j < nzc)
            def _():
                sl = pl.ds(j * ZC, ZC)
                pltpu.sync_copy(tx_hbm.at[sl, :], txv)
                pltpu.sync_copy(dinvf_hbm.at[sl, :], dvv)

                def body(i, carry):
                    r = i // nfc
                    cc = (i % nfc) * 16
                    txv[r, pl.ds(cc, 16)] = (txv[r, pl.ds(cc, 16)] *
                                             dvv[r, pl.ds(cc, 16)])
                    return carry

                lax.fori_loop(0, ZC * nfc, body, 0)
                pltpu.sync_copy(txv, u_hbm.at[sl, :])

    return mul


def _make_combine_sc(n_rows, F, alpha, beta, ZC=200):
    assert n_rows % ZC == 0 and ZC % 8 == 0 and F % 16 == 0
    NW = NCORES * NSUB
    nzc = n_rows // ZC
    trips = _cdiv(nzc, NW)
    nfc = F // 16

    @functools.partial(
        pl.kernel,
        out_type=[
            jax.ShapeDtypeStruct((n_rows, F), jnp.float32),
            jax.ShapeDtypeStruct((n_rows, F), jnp.float32),
        ],
        mesh=_sc_mesh(),
        compiler_params=_SC_PARAMS,
        scratch_types=[
            pltpu.VMEM((ZC, F), jnp.float32),
            pltpu.VMEM((ZC, F), jnp.float32),
            pltpu.VMEM((ZC, F), jnp.float32),
            pltpu.VMEM((ZC, F), jnp.float32),
        ],
    )
    def comb(p0_hbm, p1_hbm, dinvf_hbm, sub_hbm, tx_hbm, u_hbm,
             pa, pb, dv, sv):
        c = lax.axis_index("c")
        s = lax.axis_index("s")
        wid = s * NCORES + c
        for t in range(trips):
            j = t * NW + wid

            @pl.when(j < nzc)
            def _():
                sl = pl.ds(j * ZC, ZC)
                pltpu.sync_copy(p0_hbm.at[sl, :], pa)
                pltpu.sync_copy(p1_hbm.at[sl, :], pb)
                pltpu.sync_copy(dinvf_hbm.at[sl, :], dv)
                pltpu.sync_copy(sub_hbm.at[sl, :], sv)

                def body(i, carry):
                    r = i // nfc
                    cc = pl.ds((i % nfc) * 16, 16)
                    tx = (alpha * dv[r, cc] * (pa[r, cc] + pb[r, cc]) +
                          beta * sv[r, cc])
                    pa[r, cc] = tx
                    pb[r, cc] = dv[r, cc] * tx
                    return carry

                lax.fori_loop(0, ZC * nfc, body, 0)
                pltpu.sync_copy(pa, tx_hbm.at[sl, :])
                pltpu.sync_copy(pb, u_hbm.at[sl, :])

    return comb


def _combine_tc(part, dinvF, sub, alpha, beta, BN):
    _, n, F = part.shape
    assert n % BN == 0

    def body(p_ref, d_ref, s_ref, tx_ref):
        g = p_ref[0] + p_ref[1]
        tx_ref[...] = alpha * d_ref[...] * g + beta * s_ref[...]

    return pl.pallas_call(
        body,
        grid=(n // BN,),
        in_specs=[
            pl.BlockSpec((NCORES, BN, F), lambda i: (0, i, 0)),
            pl.BlockSpec((BN, F), lambda i: (i, 0)),
            pl.BlockSpec((BN, F), lambda i: (i, 0)),
        ],
        out_specs=pl.BlockSpec((BN, F), lambda i: (i, 0)),
        out_shape=jax.ShapeDtypeStruct((n, F), jnp.float32),
    )(part, dinvF, sub)

def _pre_tc(deg_part, F, BN):
    n = deg_part.shape[1]
    assert n % BN == 0

    def body(dp_ref, dinvf_ref):
        deg = dp_ref[0] + dp_ref[1]
        dinv = jnp.where(deg > 0.0, lax.rsqrt(jnp.maximum(deg, 1e-20)), 0.0)
        dinvf_ref[...] = jnp.broadcast_to(dinv, (BN, F))

    return pl.pallas_call(
        body,
        grid=(n // BN,),
        in_specs=[pl.BlockSpec((NCORES, BN, 1), lambda i: (0, i, 0))],
        out_specs=pl.BlockSpec((BN, F), lambda i: (i, 0)),
        out_shape=jax.ShapeDtypeStruct((n, F), jnp.float32),
    )(deg_part)


def _sum_tc(part, BN):
    _, n, F = part.shape
    assert n % BN == 0

    def body(p_ref, x_ref):
        x_ref[...] = p_ref[0] + p_ref[1]

    return pl.pallas_call(
        body,
        grid=(n // BN,),
        in_specs=[pl.BlockSpec((NCORES, BN, F), lambda i: (0, i, 0))],
        out_specs=pl.BlockSpec((BN, F), lambda i: (i, 0)),
        out_shape=jax.ShapeDtypeStruct((n, F), jnp.float32),
    )(part)


def _cheb_matmul_tc(txs, Wmat, b, vals, relu, BN):
    n, Fin = txs[0].shape
    Fout = Wmat.shape[2]
    assert n % BN == 0
    nv = 1 if vals is not None else 0

    def body(*refs):
        tx_refs = refs[:K]
        w_ref = refs[K]
        b_ref = refs[K + 1]
        v_ref = refs[K + 2] if nv else None
        o_ref = refs[K + 2 + nv]
        acc = jnp.zeros((BN, Fout), jnp.float32) + b_ref[...]
        for k in range(K):
            acc = acc + jnp.dot(tx_refs[k][...], w_ref[k],
                                preferred_element_type=jnp.float32)
        if relu:
            acc = jnp.maximum(acc, 0.0)
        if nv:
            acc = acc * v_ref[...]
        o_ref[...] = acc

    in_specs = [pl.BlockSpec((BN, Fin), lambda i: (i, 0)) for _ in range(K)]
    in_specs.append(pl.BlockSpec((K, Fin, Fout), lambda i: (0, 0, 0)))
    in_specs.append(pl.BlockSpec((1, Fout), lambda i: (0, 0)))
    args = list(txs) + [Wmat, b]
    if nv:
        in_specs.append(pl.BlockSpec((BN, 1), lambda i: (i, 0)))
        args.append(vals)
    return pl.pallas_call(
        body,
        grid=(n // BN,),
        in_specs=in_specs,
        out_specs=pl.BlockSpec((BN, Fout), lambda i: (i, 0)),
        out_shape=jax.ShapeDtypeStruct((n, Fout), jnp.float32),
    )(*args)


def _final_tc(Wl, xf, bl, BC):
    M = Wl.shape[1]
    assert M % BC == 0

    def body(w_ref, x_ref, b_ref, o_ref):
        i = pl.program_id(0)

        @pl.when(i == 0)
        def _():
            o_ref[...] = b_ref[...]

        o_ref[...] += jnp.dot(w_ref[...], x_ref[...],
                              preferred_element_type=jnp.float32)

    return pl.pallas_call(
        body,
        grid=(M // BC,),
        in_specs=[
            pl.BlockSpec((NUM_CLASSES, BC), lambda i: (0, i)),
            pl.BlockSpec((BC, 1), lambda i: (i, 0)),
            pl.BlockSpec((NUM_CLASSES, 1), lambda i: (0, 0)),
        ],
        out_specs=pl.BlockSpec((NUM_CLASSES, 1), lambda i: (0, 0)),
        out_shape=jax.ShapeDtypeStruct((NUM_CLASSES, 1), jnp.float32),
    )(Wl, xf, bl)


def kernel(pos, edge_index, E1_index, E2_index, D0_rows, D0_cols, D0_vals,
           D1_rows, D1_cols, D1_vals, W0, b0, W1, b1, W2, b2, Wl, bl):
    f32 = jnp.float32
    prop0 = _make_prop_sc(N0, 8, NE0, 2000, 1000)
    prop1 = _make_prop_sc(N1, 32, NE1, 2000, 1000)
    prop2 = _make_prop_sc(N2P, 64, NE2, 1000, 200)
    deg0 = _make_deg_sc(N0, NE0, 2000, 400)
    deg1 = _make_deg_sc(25600, NE1, 2000, 400)
    deg2 = _make_deg_sc(N2P, NE2, 2000, 400)
    pool0 = _make_pool_sc(N0, 32, N1, 2000, 1000)
    pool1 = _make_pool_sc(N1, 64, N2P, 1000, 200)
    mul0 = _make_mul_sc(N0 // 2, 16, ZC=1000)
    mul1 = _make_mul_sc(N1, 32, ZC=1000)
    mul2 = _make_mul_sc(N2P, 64, ZC=400)
    z8 = jnp.zeros((200, 8), f32)
    z32 = jnp.zeros((200, 32), f32)
    z64 = jnp.zeros((200, 64), f32)
    ident = lambda a: a
    vw0 = lambda a: a.reshape(N0 // 2, 16)
    uv0 = lambda a: a.reshape(N0, 8)

    def run_layer(x, src, dst, prop_fn, mul_fn, vw, uv, z,
                  deg_part, Wmat, bias, vals, relu, BN):
        Fd = x.shape[1]
        dinvF = _pre_tc(deg_part, Fd, BN)
        dFv = vw(dinvF)
        txs = [x]
        u = mul_fn(vw(x), dFv)
        for k in range(1, K):
            part = prop_fn(uv(u), src, dst, z)
            if k == 1:
                tx = _combine_tc(part, dinvF, x, -1.0, 0.0, BN)
            else:
                tx = _combine_tc(part, dinvF, txs[-2], -2.0, -1.0, BN)
            txs.append(tx)
            if k < K - 1:
                u = mul_fn(vw(tx), dFv)
        b2d = bias.reshape(1, -1)
        return _cheb_matmul_tc(txs, Wmat, b2d, vals, relu, BN)

    x0 = jnp.concatenate([pos, jnp.zeros((N0, 5), f32)], axis=1)
    W0p = jnp.concatenate([W0, jnp.zeros((K, 5, 32), f32)], axis=1)
    src0 = edge_index[0].astype(jnp.int32)
    dst0 = edge_index[1].astype(jnp.int32)
    dpart0 = deg0(dst0).reshape(NCORES, N0, 1)
    xv0 = run_layer(x0, src0, dst0, prop0, mul0, vw0, uv0,
                    z8, dpart0, W0p, b0, D0_vals.reshape(N0, 1), True, 1000)
    ppart0 = pool0(xv0, D0_rows.astype(jnp.int32))
    x1 = _sum_tc(ppart0, 1000)
    src1 = E1_index[0].astype(jnp.int32)
    dst1 = E1_index[1].astype(jnp.int32)
    dpart1 = deg1(dst1).reshape(NCORES, 25600, 1)[:, :N1]
    xv1 = run_layer(x1, src1, dst1, prop1, mul1, ident, ident,
                    z32, dpart1, W1, b1, D1_vals.reshape(N1, 1), True, 1000)
    ppart1 = pool1(xv1, D1_rows.astype(jnp.int32))
    x2 = _sum_tc(ppart1, 1280)
    src2 = E2_index[0].astype(jnp.int32)
    dst2 = E2_index[1].astype(jnp.int32)
    dpart2 = deg2(dst2).reshape(NCORES, N2P, 1)
    x3 = run_layer(x2, src2, dst2, prop2, mul2, ident, ident,
                   z64, dpart2, W2, b2, None, False, 1280)
    xf = x3[:N2].reshape(N2 * 64, 1)
    out = _final_tc(Wl, xf, bl.reshape(NUM_CLASSES, 1), 16000)
    return out.reshape(NUM_CLASSES)


# speedup vs baseline: 1.1916x; 1.0529x over previous
import functools

import jax
import jax.numpy as jnp
from jax import lax
from jax.experimental import pallas as pl
from jax.experimental.pallas import tpu as pltpu
from jax.experimental.pallas import tpu_sc as plsc

N0, N1, N2 = 100000, 25000, 6250
N2P = 6400
NE0, NE1, NE2 = 1600000, 400000, 100000
K = 6
NUM_CLASSES = 40

NCORES, NSUB = 2, 16


def _sc_mesh():
    return plsc.VectorSubcoreMesh(
        core_axis_name="c", subcore_axis_name="s",
        num_cores=NCORES, num_subcores=NSUB)


_SC_PARAMS = pltpu.CompilerParams(use_tc_tiling_on_sc=False)


def _cdiv(a, b):
    return -(-a // b)


def _make_prop_sc(n_pad, F, NE, W, ZC, idx_scale=1, ZZ=200):
    assert NE % W == 0 and W % 8 == 0 and n_pad % ZC == 0 and ZC % 8 == 0
    assert ZC <= W and n_pad % ZZ == 0 and ZZ % 8 == 0
    nwin = NE // W
    nwin_pc = _cdiv(nwin, NCORES)
    trips = _cdiv(nwin_pc, NSUB)
    nzc = n_pad // ZC
    ztrips = _cdiv(nzc, NSUB)
    nzz = n_pad // ZZ
    zztrips = _cdiv(nzz, NSUB)

    @functools.partial(
        pl.kernel,
        out_type=jax.ShapeDtypeStruct((NCORES, n_pad, F), jnp.float32),
        mesh=_sc_mesh(),
        compiler_params=_SC_PARAMS,
        scratch_types=[
            pltpu.VMEM((W,), jnp.int32),
            pltpu.VMEM((W,), jnp.int32),
            pltpu.VMEM((W,), jnp.int32),
            pltpu.VMEM((W,), jnp.int32),
            pltpu.VMEM((W, F), jnp.float32),
            pltpu.VMEM((ZZ, F), jnp.float32),
            pltpu.VMEM_SHARED((n_pad, F), jnp.float32),
            pltpu.SemaphoreType.DMA,
            pltpu.SemaphoreType.DMA,
            pltpu.SemaphoreType.DMA,
        ],
    )
    def prop(u_hbm, src_hbm, dst_hbm, z_hbm, part_hbm, idx_sa, idx_sb,
             idx_da, idx_db, rows, zb, ysh, sem, semi_a, semi_b):
        c = lax.axis_index("c")
        s = lax.axis_index("s")
        idx_s = [idx_sa, idx_sb]
        idx_d = [idx_da, idx_db]
        semi = [semi_a, semi_b]
        pltpu.sync_copy(z_hbm, zb)
        for t in range(zztrips):
            j = t * NSUB + s

            @pl.when(j < nzz)
            def _():
                pltpu.sync_copy(zb, ysh.at[pl.ds(j * ZZ, ZZ), :])
        plsc.subcore_barrier()

        def wguard(t):
            wloc = t * NSUB + s
            g = c * nwin_pc + wloc
            return (wloc < nwin_pc) & (g < nwin)

        def woff(t):
            return (c * nwin_pc + t * NSUB + s) * W

        def issue_idx(t):
            b = t % 2
            pltpu.async_copy(src_hbm.at[pl.ds(woff(t), W)], idx_s[b],
                             semi[b])
            pltpu.async_copy(dst_hbm.at[pl.ds(woff(t), W)], idx_d[b],
                             semi[b])

        @pl.when(wguard(0))
        def _():
            issue_idx(0)

        for t in range(trips):
            b = t % 2

            @pl.when(wguard(t))
            def _():
                if t + 1 < trips:
                    @pl.when(wguard(t + 1))
                    def _():
                        issue_idx(t + 1)
                pltpu.make_async_copy(src_hbm.at[pl.ds(0, W)], idx_s[b],
                                      semi[b]).wait()
                pltpu.make_async_copy(src_hbm.at[pl.ds(0, W)], idx_d[b],
                                      semi[b]).wait()
                if idx_scale != 1:
                    def obody(i, carry):
                        sl16 = pl.ds(i * 16, 16)
                        idx_s[b][sl16] = idx_s[b][sl16] * idx_scale
                        return carry

                    lax.fori_loop(0, W // 16, obody, 0)
                pltpu.async_copy(u_hbm.at[idx_s[b]], rows, sem).wait()
                pltpu.sync_copy(rows, ysh.at[idx_d[b]], add=True)
        plsc.subcore_barrier()
        for t in range(ztrips):
            j = t * NSUB + s

            @pl.when(j < nzc)
            def _():
                sl = pl.ds(j * ZC, ZC)
                pltpu.sync_copy(ysh.at[sl, :], rows.at[pl.ds(0, ZC), :])
                pltpu.sync_copy(rows.at[pl.ds(0, ZC), :],
                                part_hbm.at[c, sl, :])

    return prop


def _make_deg_sc(n_pad, NE, W, ZC):
    assert NE % W == 0 and W % 16 == 0 and n_pad % ZC == 0 and ZC % 16 == 0
    nwin = NE // W
    nwin_pc = _cdiv(nwin, NCORES)
    trips = _cdiv(nwin_pc, NSUB)
    nzc = n_pad // ZC
    ztrips = _cdiv(nzc, NSUB)

    @functools.partial(
        pl.kernel,
        out_type=jax.ShapeDtypeStruct((NCORES * n_pad,), jnp.float32),
        mesh=_sc_mesh(),
        compiler_params=_SC_PARAMS,
        scratch_types=[
            pltpu.VMEM((W,), jnp.int32),
            pltpu.VMEM((W,), jnp.float32),
            pltpu.VMEM((ZC,), jnp.float32),
            pltpu.VMEM_SHARED((n_pad,), jnp.float32),
        ],
    )
    def deg(dst_hbm, part_hbm, idx_d, ones_v, zv, ysh):
        c = lax.axis_index("c")
        s = lax.axis_index("s")

        def fill(i, carry):
            ones_v[pl.ds(i * 16, 16)] = jnp.ones((16,), jnp.float32)
            return carry

        lax.fori_loop(0, W // 16, fill, 0)

        def zfill(i, carry):
            zv[pl.ds(i * 16, 16)] = jnp.zeros((16,), jnp.float32)
            return carry

        lax.fori_loop(0, ZC // 16, zfill, 0)
        for t in range(ztrips):
            j = t * NSUB + s

            @pl.when(j < nzc)
            def _():
                pltpu.sync_copy(zv, ysh.at[pl.ds(j * ZC, ZC)])
        plsc.subcore_barrier()
        for t in range(trips):
            wloc = t * NSUB + s
            g = c * nwin_pc + wloc

            @pl.when((wloc < nwin_pc) & (g < nwin))
            def _():
                pltpu.sync_copy(dst_hbm.at[pl.ds(g * W, W)], idx_d)
                pltpu.sync_copy(ones_v, ysh.at[idx_d], add=True)
        plsc.subcore_barrier()
        for t in range(ztrips):
            j = t * NSUB + s

            @pl.when(j < nzc)
            def _():
                pltpu.sync_copy(ysh.at[pl.ds(j * ZC, ZC)], zv)
                pltpu.sync_copy(zv,
                                part_hbm.at[pl.ds(c * n_pad + j * ZC, ZC)])

    return deg


def _make_pool_sc(n_in, F, n_out_pad, W, ZC):
    assert n_in % W == 0 and W % 8 == 0 and n_out_pad % ZC == 0
    assert ZC <= W and ZC % 8 == 0 and F % 16 == 0
    nwin = n_in // W
    nwin_pc = _cdiv(nwin, NCORES)
    trips = _cdiv(nwin_pc, NSUB)
    nzc = n_out_pad // ZC
    ztrips = _cdiv(nzc, NSUB)

    @functools.partial(
        pl.kernel,
        out_type=jax.ShapeDtypeStruct((NCORES, n_out_pad, F), jnp.float32),
        mesh=_sc_mesh(),
        compiler_params=_SC_PARAMS,
        scratch_types=[
            pltpu.VMEM((W,), jnp.int32),
            pltpu.VMEM((W, F), jnp.float32),
            pltpu.VMEM_SHARED((n_out_pad, F), jnp.float32),
        ],
    )
    def pool(x_hbm, rows_hbm, part_hbm, idx_d, rows, ysh):
        c = lax.axis_index("c")
        s = lax.axis_index("s")
        nfc = F // 16

        def zfill(t, carry):
            rows[t // nfc, pl.ds((t % nfc) * 16, 16)] = jnp.zeros(
                (16,), jnp.float32)
            return carry

        lax.fori_loop(0, ZC * nfc, zfill, 0)
        for t in range(ztrips):
            j = t * NSUB + s

            @pl.when(j < nzc)
            def _():
                pltpu.sync_copy(rows.at[pl.ds(0, ZC), :],
                                ysh.at[pl.ds(j * ZC, ZC), :])
        plsc.subcore_barrier()
        for t in range(trips):
            wloc = t * NSUB + s
            g = c * nwin_pc + wloc

            @pl.when((wloc < nwin_pc) & (g < nwin))
            def _():
                off = g * W
                pltpu.sync_copy(x_hbm.at[pl.ds(off, W), :], rows)
                pltpu.sync_copy(rows_hbm.at[pl.ds(off, W)], idx_d)
                pltpu.sync_copy(rows, ysh.at[idx_d], add=True)
        plsc.subcore_barrier()
        for t in range(ztrips):
            j = t * NSUB + s

            @pl.when(j < nzc)
            def _():
                sl = pl.ds(j * ZC, ZC)
                pltpu.sync_copy(ysh.at[sl, :], rows.at[pl.ds(0, ZC), :])
                pltpu.sync_copy(rows.at[pl.ds(0, ZC), :],
                                part_hbm.at[c, sl, :])

    return pool


def _make_mul_sc(n_pad, F, ZC=200):
    assert n_pad % ZC == 0 and ZC % 8 == 0 and F % 16 == 0
    NW = NCORES * NSUB
    nzc = n_pad // ZC
    trips = _cdiv(nzc, NW)
    nfc = F // 16

    @functools.partial(
        pl.kernel,
        out_type=jax.ShapeDtypeStruct((n_pad, F), jnp.float32),
        mesh=_sc_mesh(),
        compiler_params=_SC_PARAMS,
        scratch_types=[
            pltpu.VMEM((ZC, F), jnp.float32),
            pltpu.VMEM((ZC, F), jnp.float32),
        ],
    )
    def mul(tx_hbm, dinvf_hbm, u_hbm, txv, dvv):
        c = lax.axis_index("c")
        s = lax.axis_index("s")
        wid = s * NCORES + c
        for t in range(trips):
            j = t * NW + wid

            @pl.when(j < nzc)
            def _():
                sl = pl.ds(j * ZC, ZC)
                pltpu.sync_copy(tx_hbm.at[sl, :], txv)
                pltpu.sync_copy(dinvf_hbm.at[sl, :], dvv)

                def body(i, carry):
                    r = i // nfc
                    cc = (i % nfc) * 16
                    txv[r, pl.ds(cc, 16)] = (txv[r, pl.ds(cc, 16)] *
                                             dvv[r, pl.ds(cc, 16)])
                    return carry

                lax.fori_loop(0, ZC * nfc, body, 0)
                pltpu.sync_copy(txv, u_hbm.at[sl, :])

    return mul


def _make_combine_sc(n_rows, F, alpha, beta, ZC=200):
    assert n_rows % ZC == 0 and ZC % 8 == 0 and F % 16 == 0
    NW = NCORES * NSUB
    nzc = n_rows // ZC
    trips = _cdiv(nzc, NW)
    nfc = F // 16

    @functools.partial(
        pl.kernel,
        out_type=[
            jax.ShapeDtypeStruct((n_rows, F), jnp.float32),
            jax.ShapeDtypeStruct((n_rows, F), jnp.float32),
        ],
        mesh=_sc_mesh(),
        compiler_params=_SC_PARAMS,
        scratch_types=[
            pltpu.VMEM((ZC, F), jnp.float32),
            pltpu.VMEM((ZC, F), jnp.float32),
            pltpu.VMEM((ZC, F), jnp.float32),
            pltpu.VMEM((ZC, F), jnp.float32),
        ],
    )
    def comb(p0_hbm, p1_hbm, dinvf_hbm, sub_hbm, tx_hbm, u_hbm,
             pa, pb, dv, sv):
        c = lax.axis_index("c")
        s = lax.axis_index("s")
        wid = s * NCORES + c
        for t in range(trips):
            j = t * NW + wid

            @pl.when(j < nzc)
            def _():
                sl = pl.ds(j * ZC, ZC)
                pltpu.sync_copy(p0_hbm.at[sl, :], pa)
                pltpu.sync_copy(p1_hbm.at[sl, :], pb)
                pltpu.sync_copy(dinvf_hbm.at[sl, :], dv)
                pltpu.sync_copy(sub_hbm.at[sl, :], sv)

                def body(i, carry):
                    r = i // nfc
                    cc = pl.ds((i % nfc) * 16, 16)
                    tx = (alpha * dv[r, cc] * (pa[r, cc] + pb[r, cc]) +
                          beta * sv[r, cc])
                    pa[r, cc] = tx
                    pb[r, cc] = dv[r, cc] * tx
                    return carry

                lax.fori_loop(0, ZC * nfc, body, 0)
                pltpu.sync_copy(pa, tx_hbm.at[sl, :])
                pltpu.sync_copy(pb, u_hbm.at[sl, :])

    return comb


def _combine_tc(part, dinvF, sub, alpha, beta, BN):
    _, n, F = part.shape
    assert n % BN == 0

    def body(p_ref, d_ref, s_ref, tx_ref):
        g = p_ref[0] + p_ref[1]
        tx_ref[...] = alpha * d_ref[...] * g + beta * s_ref[...]

    return pl.pallas_call(
        body,
        grid=(n // BN,),
        in_specs=[
            pl.BlockSpec((NCORES, BN, F), lambda i: (0, i, 0)),
            pl.BlockSpec((BN, F), lambda i: (i, 0)),
            pl.BlockSpec((BN, F), lambda i: (i, 0)),
        ],
        out_specs=pl.BlockSpec((BN, F), lambda i: (i, 0)),
        out_shape=jax.ShapeDtypeStruct((n, F), jnp.float32),
    )(part, dinvF, sub)

def _pre_tc(deg_part, F, BN):
    n = deg_part.shape[1]
    assert n % BN == 0

    def body(dp_ref, dinvf_ref):
        deg = dp_ref[0] + dp_ref[1]
        dinv = jnp.where(deg > 0.0, lax.rsqrt(jnp.maximum(deg, 1e-20)), 0.0)
        dinvf_ref[...] = jnp.broadcast_to(dinv, (BN, F))

    return pl.pallas_call(
        body,
        grid=(n // BN,),
        in_specs=[pl.BlockSpec((NCORES, BN, 1), lambda i: (0, i, 0))],
        out_specs=pl.BlockSpec((BN, F), lambda i: (i, 0)),
        out_shape=jax.ShapeDtypeStruct((n, F), jnp.float32),
    )(deg_part)


def _sum_tc(part, BN):
    _, n, F = part.shape
    assert n % BN == 0

    def body(p_ref, x_ref):
        x_ref[...] = p_ref[0] + p_ref[1]

    return pl.pallas_call(
        body,
        grid=(n // BN,),
        in_specs=[pl.BlockSpec((NCORES, BN, F), lambda i: (0, i, 0))],
        out_specs=pl.BlockSpec((BN, F), lambda i: (i, 0)),
        out_shape=jax.ShapeDtypeStruct((n, F), jnp.float32),
    )(part)


def _cheb_matmul_tc(txs, Wmat, b, vals, relu, BN):
    n, Fin = txs[0].shape
    Fout = Wmat.shape[2]
    assert n % BN == 0
    nv = 1 if vals is not None else 0

    def body(*refs):
        tx_refs = refs[:K]
        w_ref = refs[K]
        b_ref = refs[K + 1]
        v_ref = refs[K + 2] if nv else None
        o_ref = refs[K + 2 + nv]
        acc = jnp.zeros((BN, Fout), jnp.float32) + b_ref[...]
        for k in range(K):
            acc = acc + jnp.dot(tx_refs[k][...], w_ref[k],
                                preferred_element_type=jnp.float32)
        if relu:
            acc = jnp.maximum(acc, 0.0)
        if nv:
            acc = acc * v_ref[...]
        o_ref[...] = acc

    in_specs = [pl.BlockSpec((BN, Fin), lambda i: (i, 0)) for _ in range(K)]
    in_specs.append(pl.BlockSpec((K, Fin, Fout), lambda i: (0, 0, 0)))
    in_specs.append(pl.BlockSpec((1, Fout), lambda i: (0, 0)))
    args = list(txs) + [Wmat, b]
    if nv:
        in_specs.append(pl.BlockSpec((BN, 1), lambda i: (i, 0)))
        args.append(vals)
    return pl.pallas_call(
        body,
        grid=(n // BN,),
        in_specs=in_specs,
        out_specs=pl.BlockSpec((BN, Fout), lambda i: (i, 0)),
        out_shape=jax.ShapeDtypeStruct((n, Fout), jnp.float32),
    )(*args)


def _final_tc(Wl, xf, bl, BC):
    M = Wl.shape[1]
    assert M % BC == 0

    def body(w_ref, x_ref, b_ref, o_ref):
        i = pl.program_id(0)

        @pl.when(i == 0)
        def _():
            o_ref[...] = b_ref[...]

        o_ref[...] += jnp.dot(w_ref[...], x_ref[...],
                              preferred_element_type=jnp.float32)

    return pl.pallas_call(
        body,
        grid=(M // BC,),
        in_specs=[
            pl.BlockSpec((NUM_CLASSES, BC), lambda i: (0, i)),
            pl.BlockSpec((BC, 1), lambda i: (i, 0)),
            pl.BlockSpec((NUM_CLASSES, 1), lambda i: (0, 0)),
        ],
        out_specs=pl.BlockSpec((NUM_CLASSES, 1), lambda i: (0, 0)),
        out_shape=jax.ShapeDtypeStruct((NUM_CLASSES, 1), jnp.float32),
    )(Wl, xf, bl)


def kernel(pos, edge_index, E1_index, E2_index, D0_rows, D0_cols, D0_vals,
           D1_rows, D1_cols, D1_vals, W0, b0, W1, b1, W2, b2, Wl, bl):
    f32 = jnp.float32
    prop0 = _make_prop_sc(N0, 8, NE0, 2000, 1000)
    prop1 = _make_prop_sc(N1, 32, NE1, 2000, 1000)
    prop2 = _make_prop_sc(N2P, 64, NE2, 1000, 200)
    deg0 = _make_deg_sc(N0, NE0, 2000, 400)
    deg1 = _make_deg_sc(25600, NE1, 2000, 400)
    deg2 = _make_deg_sc(N2P, NE2, 2000, 400)
    pool0 = _make_pool_sc(N0, 32, N1, 2000, 1000)
    pool1 = _make_pool_sc(N1, 64, N2P, 1000, 200)
    mul0 = _make_mul_sc(N0 // 2, 16, ZC=1000)
    mul1 = _make_mul_sc(N1, 32, ZC=1000)
    mul2 = _make_mul_sc(N2P, 64, ZC=400)
    z8 = jnp.zeros((200, 8), f32)
    z32 = jnp.zeros((200, 32), f32)
    z64 = jnp.zeros((200, 64), f32)
    ident = lambda a: a
    vw0 = lambda a: a.reshape(N0 // 2, 16)
    uv0 = lambda a: a.reshape(N0, 8)

    def run_layer(x, src, dst, prop_fn, mul_fn, vw, uv, z,
                  deg_part, Wmat, bias, vals, relu, BN):
        Fd = x.shape[1]
        dinvF = _pre_tc(deg_part, Fd, BN)
        dFv = vw(dinvF)
        txs = [x]
        u = mul_fn(vw(x), dFv)
        for k in range(1, K):
            part = prop_fn(uv(u), src, dst, z)
            if k == 1:
                tx = _combine_tc(part, dinvF, x, -1.0, 0.0, BN)
            else:
                tx = _combine_tc(part, dinvF, txs[-2], -2.0, -1.0, BN)
            txs.append(tx)
            if k < K - 1:
                u = mul_fn(vw(tx), dFv)
        b2d = bias.reshape(1, -1)
        return _cheb_matmul_tc(txs, Wmat, b2d, vals, relu, BN)

    x0 = jnp.concatenate([pos, jnp.zeros((N0, 5), f32)], axis=1)
    W0p = jnp.concatenate([W0, jnp.zeros((K, 5, 32), f32)], axis=1)
    src0 = edge_index[0].astype(jnp.int32)
    dst0 = edge_index[1].astype(jnp.int32)
    dpart0 = deg0(dst0).reshape(NCORES, N0, 1)
    xv0 = run_layer(x0, src0, dst0, prop0, mul0, vw0, uv0,
                    z8, dpart0, W0p, b0, D0_vals.reshape(N0, 1), True, 1000)
    ppart0 = pool0(xv0, D0_rows.astype(jnp.int32))
    x1 = _sum_tc(ppart0, 1000)
    src1 = E1_index[0].astype(jnp.int32)
    dst1 = E1_index[1].astype(jnp.int32)
    dpart1 = deg1(dst1).reshape(NCORES, 25600, 1)[:, :N1]
    xv1 = run_layer(x1, src1, dst1, prop1, mul1, ident, ident,
                    z32, dpart1, W1, b1, D1_vals.reshape(N1, 1), True, 1000)
    ppart1 = pool1(xv1, D1_rows.astype(jnp.int32))
    x2 = _sum_tc(ppart1, 1280)
    src2 = E2_index[0].astype(jnp.int32)
    dst2 = E2_index[1].astype(jnp.int32)
    dpart2 = deg2(dst2).reshape(NCORES, N2P, 1)
    x3 = run_layer(x2, src2, dst2, prop2, mul2, ident, ident,
                   z64, dpart2, W2, b2, None, False, 1280)
    xf = x3[:N2].reshape(N2 * 64, 1)
    out = _final_tc(Wl, xf, bl.reshape(NUM_CLASSES, 1), 16000)
    return out.reshape(NUM_CLASSES)
